# Initial kernel scaffold; baseline (speedup 1.0000x reference)
import jax
import jax.numpy as jnp
from jax.experimental import pallas as pl


def kernel(pred, Laplace_W_c, Laplace_L_c, image_spiex, class_num):
    raise NotImplementedError("write your pallas kernel here")


# speedup vs baseline: 33.1557x; 33.1557x over previous
import functools

import jax
import jax.numpy as jnp
from jax import lax
from jax.experimental import pallas as pl
from jax.experimental.pallas import tpu as pltpu
from jax.experimental.pallas import tpu_sc as plsc

B = 4
C = 21
S = 1200
N = 512 * 512
NW = 32
NCHUNK = 16
CHUNK = N // NCHUNK
NTASK = B * NCHUNK
REPS = NTASK // NW
ACC = C * S
STEPS = CHUNK // 16


def _sc_pool_body(spx_hbm, predr_hbm, partials_hbm, sums_hbm,
                  seg_v, val_v, acc_v, ssum_v):
    wid = lax.axis_index("s") * 2 + lax.axis_index("c")
    for rep in range(REPS):
        t = wid + rep * NW
        b = t // NCHUNK
        off = (t % NCHUNK) * CHUNK

        pltpu.sync_copy(spx_hbm.at[pl.ds(b * N + off, CHUNK)], seg_v)

        def zero_body(i, carry):
            acc_v[pl.ds(i * 16, 16)] = jnp.zeros((16,), jnp.float32)
            return carry
        lax.fori_loop(0, ACC // 16, zero_body, 0)

        def sum_body(i, s):
            return s + seg_v[pl.ds(i * 16, 16)]
        ssum_v[...] = lax.fori_loop(
            0, STEPS, sum_body, jnp.zeros((16,), jnp.int32))
        pltpu.sync_copy(ssum_v, sums_hbm.at[pl.ds(t * 16, 16)])

        for c in range(C):
            pltpu.sync_copy(
                predr_hbm.at[pl.ds((b * C + c) * N + off, CHUNK)], val_v)
            base = jnp.full((16,), c * S, jnp.int32)

            def scat_body(i, carry):
                seg = seg_v[pl.ds(i * 16, 16)]
                vals = val_v[pl.ds(i * 16, 16)]
                plsc.addupdate_scatter(acc_v, [seg + base], vals)
                return carry
            lax.fori_loop(0, STEPS, scat_body, 0, unroll=2)

        pltpu.sync_copy(acc_v, partials_hbm.at[pl.ds(t * ACC, ACC)])


_sc_pool = functools.partial(
    pl.kernel,
    out_type=[
        jax.ShapeDtypeStruct((NTASK * ACC,), jnp.float32),
        jax.ShapeDtypeStruct((NTASK * 16,), jnp.int32),
    ],
    mesh=plsc.VectorSubcoreMesh(core_axis_name="c", subcore_axis_name="s"),
    compiler_params=pltpu.CompilerParams(needs_layout_passes=False),
    scratch_types=[
        pltpu.VMEM((CHUNK,), jnp.int32),
        pltpu.VMEM((CHUNK,), jnp.float32),
        pltpu.VMEM((ACC,), jnp.float32),
        pltpu.VMEM((16,), jnp.int32),
    ],
)(_sc_pool_body)


def _tc_finish_body(partials_ref, sums_ref, l_ref, w_ref, out_ref,
                    r_v, qacc_v, wacc_v):
    b = pl.program_id(0)

    @pl.when(b == 0)
    def _init():
        qacc_v[...] = jnp.zeros_like(qacc_v)
        wacc_v[0, 0] = 0.0

    denom = jnp.sum(sums_ref[...]).astype(jnp.float32) + 1e-16
    r_v[...] = jnp.sum(partials_ref[0], axis=0) / denom

    wb = w_ref[0]
    wacc_v[0, 0] += jnp.sum(wb * wb)

    r = r_v[...]
    y = lax.dot_general(r, l_ref[0], (((1,), (0,)), ((), ())),
                        precision=lax.Precision.HIGHEST,
                        preferred_element_type=jnp.float32)
    qacc_v[...] += jnp.sum(y * r, axis=1, keepdims=True)

    @pl.when(b == B - 1)
    def _fin():
        out_ref[0, 0] = (2.0 / jnp.sqrt(wacc_v[0, 0])) * (
            jnp.sum(qacc_v[...]) / C)


_tc_finish = pl.pallas_call(
    _tc_finish_body,
    grid=(B,),
    in_specs=[
        pl.BlockSpec((1, NCHUNK, C, S), lambda b: (b, 0, 0, 0)),
        pl.BlockSpec((1, NCHUNK, 16), lambda b: (b, 0, 0)),
        pl.BlockSpec((1, S, S), lambda b: (b, 0, 0)),
        pl.BlockSpec((1, S, S), lambda b: (b, 0, 0)),
    ],
    out_specs=pl.BlockSpec(memory_space=pltpu.SMEM),
    out_shape=jax.ShapeDtypeStruct((1, 1), jnp.float32),
    scratch_shapes=[
        pltpu.VMEM((C, S), jnp.float32),
        pltpu.VMEM((C, 1), jnp.float32),
        pltpu.SMEM((1, 1), jnp.float32),
    ],
)


def kernel(pred, Laplace_W_c, Laplace_L_c, image_spiex, class_num):
    predr = jnp.transpose(pred, (0, 2, 3, 1)).reshape(B * C * N)
    spx = image_spiex.reshape(B * N).astype(jnp.int32)
    partials, sums = _sc_pool(spx, predr)
    out = _tc_finish(
        partials.reshape(B, NCHUNK, C, S),
        sums.reshape(B, NCHUNK, 16),
        Laplace_L_c.astype(jnp.float32),
        Laplace_W_c.astype(jnp.float32),
    )
    return out[0, 0] + jnp.asarray(class_num - C, jnp.float32)


# speedup vs baseline: 40.9924x; 1.2364x over previous
import functools

import jax
import jax.numpy as jnp
from jax import lax
from jax.experimental import pallas as pl
from jax.experimental.pallas import tpu as pltpu
from jax.experimental.pallas import tpu_sc as plsc

B = 4
C = 21
S = 1200
N = 512 * 512
NW = 32
NCHUNK = 16
CHUNK = N // NCHUNK
NTASK = B * NCHUNK
REPS = NTASK // NW
ACC = C * S
STEPS = CHUNK // 16


def _sc_pool_body(spx_hbm, predr_hbm, partials_hbm, sums_hbm,
                  seg_v, val0_v, val1_v, acc_v, ssum_v, sem0, sem1):
    wid = lax.axis_index("s") * 2 + lax.axis_index("c")
    bufs = (val0_v, val1_v)
    sems = (sem0, sem1)

    def val_copy(b, off, c, buf, sem):
        return pltpu.async_copy(
            predr_hbm.at[pl.ds((b * C + c) * N + off, CHUNK)], buf, sem)

    for rep in range(REPS):
        t = wid + rep * NW
        b = t // NCHUNK
        off = (t % NCHUNK) * CHUNK

        cp = val_copy(b, off, 0, bufs[0], sems[0])
        pltpu.sync_copy(spx_hbm.at[pl.ds(b * N + off, CHUNK)], seg_v)

        def zero_body(i, carry):
            acc_v[pl.ds(i * 16, 16)] = jnp.zeros((16,), jnp.float32)
            return carry
        lax.fori_loop(0, ACC // 16, zero_body, 0)

        def sum_body(i, s):
            return s + seg_v[pl.ds(i * 16, 16)]
        ssum_v[...] = lax.fori_loop(
            0, STEPS, sum_body, jnp.zeros((16,), jnp.int32))
        pltpu.sync_copy(ssum_v, sums_hbm.at[pl.ds(t * 16, 16)])

        for c in range(C):
            val_v = bufs[c % 2]
            cp.wait()
            if c + 1 < C:
                cp = val_copy(b, off, c + 1, bufs[(c + 1) % 2],
                              sems[(c + 1) % 2])
            base = jnp.full((16,), c * S, jnp.int32)

            def scat_body(i, carry):
                seg, vals = carry
                nxt = jnp.minimum(i + 1, STEPS - 1) * 16
                seg_n = seg_v[pl.ds(nxt, 16)]
                val_n = val_v[pl.ds(nxt, 16)]
                plsc.addupdate_scatter(acc_v, [seg + base], vals)
                return (seg_n, val_n)
            first = (seg_v[pl.ds(0, 16)], val_v[pl.ds(0, 16)])
            lax.fori_loop(0, STEPS, scat_body, first, unroll=4)

        pltpu.sync_copy(acc_v, partials_hbm.at[pl.ds(t * ACC, ACC)])


_sc_pool = functools.partial(
    pl.kernel,
    out_type=[
        jax.ShapeDtypeStruct((NTASK * ACC,), jnp.float32),
        jax.ShapeDtypeStruct((NTASK * 16,), jnp.int32),
    ],
    mesh=plsc.VectorSubcoreMesh(core_axis_name="c", subcore_axis_name="s"),
    compiler_params=pltpu.CompilerParams(needs_layout_passes=False),
    scratch_types=[
        pltpu.VMEM((CHUNK,), jnp.int32),
        pltpu.VMEM((CHUNK,), jnp.float32),
        pltpu.VMEM((CHUNK,), jnp.float32),
        pltpu.VMEM((ACC,), jnp.float32),
        pltpu.VMEM((16,), jnp.int32),
        pltpu.SemaphoreType.DMA,
        pltpu.SemaphoreType.DMA,
    ],
)(_sc_pool_body)


def _tc_finish_body(partials_ref, sums_ref, l_ref, w_ref, out_ref,
                    r_v, qacc_v, wacc_v):
    b = pl.program_id(0)

    @pl.when(b == 0)
    def _init():
        qacc_v[...] = jnp.zeros_like(qacc_v)
        wacc_v[0, 0] = 0.0

    denom = jnp.sum(sums_ref[...]).astype(jnp.float32) + 1e-16
    r_v[...] = jnp.sum(partials_ref[0], axis=0) / denom

    wb = w_ref[0]
    wacc_v[0, 0] += jnp.sum(wb * wb)

    r = r_v[...]
    y = lax.dot_general(r, l_ref[0], (((1,), (0,)), ((), ())),
                        precision=lax.Precision.HIGHEST,
                        preferred_element_type=jnp.float32)
    qacc_v[...] += jnp.sum(y * r, axis=1, keepdims=True)

    @pl.when(b == B - 1)
    def _fin():
        out_ref[0, 0] = (2.0 / jnp.sqrt(wacc_v[0, 0])) * (
            jnp.sum(qacc_v[...]) / C)


_tc_finish = pl.pallas_call(
    _tc_finish_body,
    grid=(B,),
    in_specs=[
        pl.BlockSpec((1, NCHUNK, C, S), lambda b: (b, 0, 0, 0)),
        pl.BlockSpec((1, NCHUNK, 16), lambda b: (b, 0, 0)),
        pl.BlockSpec((1, S, S), lambda b: (b, 0, 0)),
        pl.BlockSpec((1, S, S), lambda b: (b, 0, 0)),
    ],
    out_specs=pl.BlockSpec(memory_space=pltpu.SMEM),
    out_shape=jax.ShapeDtypeStruct((1, 1), jnp.float32),
    scratch_shapes=[
        pltpu.VMEM((C, S), jnp.float32),
        pltpu.VMEM((C, 1), jnp.float32),
        pltpu.SMEM((1, 1), jnp.float32),
    ],
)


def kernel(pred, Laplace_W_c, Laplace_L_c, image_spiex, class_num):
    predr = jnp.transpose(pred, (0, 2, 3, 1)).reshape(B * C * N)
    spx = image_spiex.reshape(B * N).astype(jnp.int32)
    partials, sums = _sc_pool(spx, predr)
    out = _tc_finish(
        partials.reshape(B, NCHUNK, C, S),
        sums.reshape(B, NCHUNK, 16),
        Laplace_L_c.astype(jnp.float32),
        Laplace_W_c.astype(jnp.float32),
    )
    return out[0, 0] + jnp.asarray(class_num - C, jnp.float32)
